# baseline (speedup 1.0000x reference)
import functools

import jax
import jax.numpy as jnp
from jax import lax
from jax.experimental import pallas as pl
from jax.experimental.pallas import tpu as pltpu
from jax.experimental.pallas import tpu_sc as plsc

E = 320000
D = 128
N_SEG = 10000
N_PAD = 10240

N_CORES = 2
N_SUB = 16
N_WORK = N_CORES * N_SUB
TPW = E // N_WORK
ROW = 80
ROWS_PT = TPW // ROW

BE = 16384
BO = BE // D


def _tc_body(x_ref, w_ref, b_ref, o_ref):
    w2 = jnp.broadcast_to(w_ref[...], (D, D))
    r = lax.dot_general(
        x_ref[...], w2, (((1,), (0,)), ((), ())),
        preferred_element_type=jnp.float32,
    )
    r3 = r.reshape(BO, D, D)
    mask = (lax.broadcasted_iota(jnp.int32, (D, D), 0)
            == lax.broadcasted_iota(jnp.int32, (D, D), 1)).astype(jnp.float32)
    lat = jnp.sum(r3 * mask[None], axis=1) + b_ref[0, 0]
    lat = jnp.where(lat >= 0, lat, 0.2 * lat)
    o_ref[...] = jnp.exp(lat)


def _tc_exp_latent(x, w_col, b):
    grid = pl.cdiv(E, BE)
    return pl.pallas_call(
        _tc_body,
        grid=(grid,),
        in_specs=[
            pl.BlockSpec((BE, D), lambda i: (i, 0)),
            pl.BlockSpec((D, 1), lambda i: (0, 0)),
            pl.BlockSpec((1, 1), lambda i: (0, 0)),
        ],
        out_specs=pl.BlockSpec((BO, D), lambda i: (i, 0)),
        out_shape=jax.ShapeDtypeStruct((E // D, D), jnp.float32),
    )(x, w_col, b)


K_PIPE = 10
SLICE = N_PAD // N_SUB
SPW = E // N_SUB
SROWS = SPW // ROW


def _sc_body(ex_hbm, idx2_hbm, idxf_hbm, out_hbm,
             ex_v, idx2_v, idxf_v, gat_v, out_v, zero_v, seg_sh, sem):
    c = lax.axis_index("c")
    s = lax.axis_index("s")
    sbase = pl.multiple_of(s * SPW, SPW)
    coff = pl.multiple_of(c * TPW, TPW)
    gbase = pl.multiple_of(s * SPW + c * TPW, TPW)
    pltpu.sync_copy(ex_hbm.at[pl.ds(sbase, SPW)], ex_v)
    pltpu.sync_copy(idx2_hbm.at[s], idx2_v)
    pltpu.sync_copy(idxf_hbm.at[pl.ds(gbase, TPW)], idxf_v)

    zero16 = jnp.zeros((16,), jnp.float32)

    def zbody(i, cc):
        zero_v[pl.ds(pl.multiple_of(i * 16, 16), 16)] = zero16
        return cc

    lax.fori_loop(0, SLICE // 16, zbody, 0)
    zbase = pl.multiple_of(s * SLICE, SLICE)
    pltpu.sync_copy(zero_v, seg_sh.at[pl.ds(zbase, SLICE)])
    plsc.subcore_barrier()

    def sbody(j, cc):
        descs = []
        for t in range(K_PIPE):
            row = j * K_PIPE + t
            src = ex_v.at[pl.ds(pl.multiple_of(row * ROW, ROW), ROW)]
            descs.append(
                pltpu.async_copy(src, seg_sh.at[idx2_v.at[row]], sem, add=True))
        for d in descs:
            d.wait()
        return cc

    lax.fori_loop(0, SROWS // K_PIPE, sbody, 0)
    plsc.subcore_barrier()

    pltpu.sync_copy(seg_sh.at[idxf_v], gat_v)

    def dbody(j, cc):
        off = pl.multiple_of(j * 16, 16)
        eoff = pl.multiple_of(c * TPW + j * 16, 16)
        out_v[pl.ds(off, 16)] = ex_v[pl.ds(eoff, 16)] / gat_v[pl.ds(off, 16)]
        return cc

    lax.fori_loop(0, TPW // 16, dbody, 0)
    pltpu.sync_copy(out_v, out_hbm.at[pl.ds(gbase, TPW)])


def _sc_softmax(ex, idx2, idxf):
    mesh = plsc.VectorSubcoreMesh(core_axis_name="c", subcore_axis_name="s")
    return pl.kernel(
        _sc_body,
        out_type=jax.ShapeDtypeStruct((E,), jnp.float32),
        mesh=mesh,
        scratch_types=[
            pltpu.VMEM((SPW,), jnp.float32),
            pltpu.VMEM((SROWS, ROW), jnp.int32),
            pltpu.VMEM((TPW,), jnp.int32),
            pltpu.VMEM((TPW,), jnp.float32),
            pltpu.VMEM((TPW,), jnp.float32),
            pltpu.VMEM((SLICE,), jnp.float32),
            pltpu.VMEM_SHARED((N_PAD,), jnp.float32),
            pltpu.SemaphoreType.DMA,
        ],
    )(ex, idx2, idxf)


def kernel(input, index, W, b):
    ex = _tc_exp_latent(input, W.reshape(D, 1), b.reshape(1, 1)).reshape(E)
    idx2 = index.reshape(N_SUB, SROWS, ROW)
    out = _sc_softmax(ex, idx2, index)
    return out.reshape(E, 1)


# speedup vs baseline: 1.0005x; 1.0005x over previous
import jax
import jax.numpy as jnp
from jax import lax
from jax.experimental import pallas as pl
from jax.experimental.pallas import tpu as pltpu
from jax.experimental.pallas import tpu_sc as plsc

E = 320000
D = 128
N_SEG = 10000
N_PAD = 10240

N_CORES = 2
N_SUB = 16
N_WORK = N_CORES * N_SUB
TPW = E // N_WORK
ROW = 80
ROWS_PT = TPW // ROW

BE = 16384
BO = BE // D


def _tc_body(x_ref, w_ref, b_ref, o_ref):
    w2 = jnp.broadcast_to(w_ref[...], (D, D))
    r = lax.dot_general(
        x_ref[...], w2, (((1,), (0,)), ((), ())),
        preferred_element_type=jnp.float32,
    )
    r3 = r.reshape(BO, D, D)
    mask = (lax.broadcasted_iota(jnp.int32, (D, D), 0)
            == lax.broadcasted_iota(jnp.int32, (D, D), 1)).astype(jnp.float32)
    lat = jnp.sum(r3 * mask[None], axis=1) + b_ref[0, 0]
    lat = jnp.where(lat >= 0, lat, 0.2 * lat)
    o_ref[...] = jnp.exp(lat)


def _tc_exp_latent(x, w_col, b):
    grid = pl.cdiv(E, BE)
    return pl.pallas_call(
        _tc_body,
        grid=(grid,),
        in_specs=[
            pl.BlockSpec((BE, D), lambda i: (i, 0)),
            pl.BlockSpec((D, 1), lambda i: (0, 0)),
            pl.BlockSpec((1, 1), lambda i: (0, 0)),
        ],
        out_specs=pl.BlockSpec((BO, D), lambda i: (i, 0)),
        out_shape=jax.ShapeDtypeStruct((E // D, D), jnp.float32),
    )(x, w_col, b)


K_PIPE = 10
SLICE = N_PAD // N_SUB
SPW = E // N_SUB
SROWS = SPW // ROW


def _sc_body(ex_hbm, idx2_hbm, idxf_hbm, out_hbm,
             ex_v, idx2_v, idxf_v, gat_v, out_v, zero_v, seg_sh, sem):
    c = lax.axis_index("c")
    s = lax.axis_index("s")
    sbase = pl.multiple_of(s * SPW, SPW)
    coff = pl.multiple_of(c * TPW, TPW)
    gbase = pl.multiple_of(s * SPW + c * TPW, TPW)
    pltpu.sync_copy(ex_hbm.at[pl.ds(sbase, SPW)], ex_v)
    pltpu.sync_copy(idx2_hbm.at[s], idx2_v)
    pltpu.sync_copy(idxf_hbm.at[pl.ds(gbase, TPW)], idxf_v)

    zero16 = jnp.zeros((16,), jnp.float32)

    def zbody(i, cc):
        zero_v[pl.ds(pl.multiple_of(i * 16, 16), 16)] = zero16
        return cc

    lax.fori_loop(0, SLICE // 16, zbody, 0)
    zbase = pl.multiple_of(s * SLICE, SLICE)
    pltpu.sync_copy(zero_v, seg_sh.at[pl.ds(zbase, SLICE)])
    plsc.subcore_barrier()

    def sbody(j, cc):
        descs = []
        for t in range(K_PIPE):
            row = j * K_PIPE + t
            src = ex_v.at[pl.ds(pl.multiple_of(row * ROW, ROW), ROW)]
            descs.append(
                pltpu.async_copy(src, seg_sh.at[idx2_v.at[row]], sem, add=True))
        for d in descs:
            d.wait()
        return cc

    lax.fori_loop(0, SROWS // K_PIPE, sbody, 0)
    plsc.subcore_barrier()

    pltpu.sync_copy(seg_sh.at[idxf_v], gat_v)

    def dbody(j, cc):
        off = pl.multiple_of(j * 16, 16)
        eoff = pl.multiple_of(c * TPW + j * 16, 16)
        out_v[pl.ds(off, 16)] = ex_v[pl.ds(eoff, 16)] / gat_v[pl.ds(off, 16)]
        return cc

    lax.fori_loop(0, TPW // 16, dbody, 0)
    pltpu.sync_copy(out_v, out_hbm.at[pl.ds(gbase, TPW)])


def _sc_softmax(ex, idx2, idxf):
    mesh = plsc.VectorSubcoreMesh(core_axis_name="c", subcore_axis_name="s")
    return pl.kernel(
        _sc_body,
        out_type=jax.ShapeDtypeStruct((E,), jnp.float32),
        mesh=mesh,
        scratch_types=[
            pltpu.VMEM((SPW,), jnp.float32),
            pltpu.VMEM((SROWS, ROW), jnp.int32),
            pltpu.VMEM((TPW,), jnp.int32),
            pltpu.VMEM((TPW,), jnp.float32),
            pltpu.VMEM((TPW,), jnp.float32),
            pltpu.VMEM((SLICE,), jnp.float32),
            pltpu.VMEM_SHARED((N_PAD,), jnp.float32),
            pltpu.SemaphoreType.DMA,
        ],
    )(ex, idx2, idxf)


def kernel(input, index, W, b):
    ex = _tc_exp_latent(input, W.reshape(D, 1), b.reshape(1, 1)).reshape(E)
    idx2 = index.reshape(N_SUB, SROWS, ROW)
    out = _sc_softmax(ex, idx2, index)
    return out.reshape(E, 1)
